# Initial kernel scaffold; baseline (speedup 1.0000x reference)
import jax
import jax.numpy as jnp
from jax.experimental import pallas as pl


def kernel(x, split_features, split_thresholds, leaf_probabilities):
    raise NotImplementedError("write your pallas kernel here")


# speedup vs baseline: 45.1108x; 45.1108x over previous
import functools

import jax
import jax.numpy as jnp
from jax import lax
from jax.experimental import pallas as pl
from jax.experimental.pallas import tpu as pltpu
from jax.experimental.pallas import tpu_sc as plsc

INPUT_DIM = 512
N_CLASSES = 128
MAX_DEPTH = 12
N_NODES = 2**MAX_DEPTH - 1
N_LEAVES = 2**MAX_DEPTH
BATCH = 16384

NC = 2
NS = 16
L = 16
NW = NC * NS
ROWS_PER_W = BATCH // NW
CHUNK = 128
NCHUNK = ROWS_PER_W // CHUNK
SUB = CHUNK // L


def _prep_body(sf_ref, lp_ref, nf_ref, table_ref):
    sf = sf_ref[...]
    nf_ref[...] = jnp.clip(jnp.floor(sf), 0, INPUT_DIM - 1).astype(jnp.int32)
    p = lp_ref[...]
    m = jnp.max(p, axis=1, keepdims=True)
    e = jnp.exp(p - m)
    table_ref[...] = e / jnp.sum(e, axis=1, keepdims=True)


def _prep(sf2d, leaf_probabilities):
    return pl.pallas_call(
        _prep_body,
        out_shape=[
            jax.ShapeDtypeStruct((N_LEAVES // 128, 128), jnp.int32),
            jax.ShapeDtypeStruct((N_LEAVES, N_CLASSES), jnp.float32),
        ],
    )(sf2d, leaf_probabilities)


def _traverse_body(
    x_ref, nf_ref, th_ref, table_ref, out_ref,
    nf_v, th_v, idx_v, gidx_v, xg_v, thr_c, rows_v,
    sem0, sem1, sem2, sem3,
):
    sems = [sem0, sem1, sem2, sem3]
    wid = lax.axis_index("s") * NC + lax.axis_index("c")
    base = wid * ROWS_PER_W

    pltpu.sync_copy(nf_ref, nf_v)
    pltpu.sync_copy(th_ref, th_v)

    lane = lax.iota(jnp.int32, L)
    zero = jnp.zeros((L,), jnp.int32)

    for c in range(NCHUNK):
        for s in range(SUB):
            off = c * CHUNK + s * L
            sl = pl.ds(s * L, L)
            idx_v[pl.ds(off, L)] = zero
            feat = plsc.load_gather(nf_v, [zero])
            thr = plsc.load_gather(th_v, [zero])
            rows = (base + off) + lane
            gidx_v[c, sl] = rows * INPUT_DIM + feat
            thr_c[c, sl] = thr
        pltpu.async_copy(x_ref.at[gidx_v.at[c]], xg_v.at[c], sems[c])

    def depth_body(d, carry):
        del d
        for c in range(NCHUNK):
            pltpu.make_async_copy(
                x_ref.at[gidx_v.at[c]], xg_v.at[c], sems[c]
            ).wait()
            for s in range(SUB):
                off = c * CHUNK + s * L
                sl = pl.ds(s * L, L)
                xv = xg_v[c, sl]
                thrv = thr_c[c, sl]
                old = idx_v[pl.ds(off, L)]
                dec = jnp.where(xv > thrv, 1, 0).astype(jnp.int32)
                new = old * 2 + 1 + dec
                idx_v[pl.ds(off, L)] = new
                feat = plsc.load_gather(nf_v, [new])
                thr = plsc.load_gather(th_v, [new])
                rows = (base + off) + lane
                gidx_v[c, sl] = rows * INPUT_DIM + feat
                thr_c[c, sl] = thr
            pltpu.async_copy(x_ref.at[gidx_v.at[c]], xg_v.at[c], sems[c])
        return carry

    lax.fori_loop(0, MAX_DEPTH - 1, depth_body, 0)

    for c in range(NCHUNK):
        pltpu.make_async_copy(
            x_ref.at[gidx_v.at[c]], xg_v.at[c], sems[c]
        ).wait()
        for s in range(SUB):
            off = c * CHUNK + s * L
            sl = pl.ds(s * L, L)
            xv = xg_v[c, sl]
            thrv = thr_c[c, sl]
            old = idx_v[pl.ds(off, L)]
            dec = jnp.where(xv > thrv, 1, 0).astype(jnp.int32)
            leaf = old * 2 + 1 + dec - N_NODES
            gidx_v[c, sl] = leaf
        pltpu.async_copy(table_ref.at[gidx_v.at[c]], rows_v.at[c], sems[c])

    for c in range(NCHUNK):
        pltpu.make_async_copy(
            table_ref.at[gidx_v.at[c]], rows_v.at[c], sems[c]
        ).wait()
        pltpu.sync_copy(
            rows_v.at[c], out_ref.at[pl.ds(base + c * CHUNK, CHUNK), :]
        )


@functools.partial(
    pl.kernel,
    out_type=jax.ShapeDtypeStruct((BATCH, N_CLASSES), jnp.float32),
    mesh=plsc.VectorSubcoreMesh(
        core_axis_name="c", subcore_axis_name="s", num_cores=NC,
        num_subcores=NS,
    ),
    scratch_types=[
        pltpu.VMEM((N_LEAVES,), jnp.int32),
        pltpu.VMEM((N_LEAVES,), jnp.float32),
        pltpu.VMEM((ROWS_PER_W,), jnp.int32),
        pltpu.VMEM((NCHUNK, CHUNK), jnp.int32),
        pltpu.VMEM((NCHUNK, CHUNK), jnp.float32),
        pltpu.VMEM((NCHUNK, CHUNK), jnp.float32),
        pltpu.VMEM((NCHUNK, CHUNK, N_CLASSES), jnp.float32),
        pltpu.SemaphoreType.DMA,
        pltpu.SemaphoreType.DMA,
        pltpu.SemaphoreType.DMA,
        pltpu.SemaphoreType.DMA,
    ],
    compiler_params=pltpu.CompilerParams(needs_layout_passes=False),
)
def _traverse(x_flat, nf, th, table, out, *scratch):
    _traverse_body(x_flat, nf, th, table, out, *scratch)


def kernel(x, split_features, split_thresholds, leaf_probabilities):
    sf2d = jnp.pad(split_features, (0, 1)).reshape(N_LEAVES // 128, 128)
    th = jnp.pad(split_thresholds, (0, 1))
    nf2d, table = _prep(sf2d, leaf_probabilities)
    nf = nf2d.reshape(N_LEAVES)
    return _traverse(x.reshape(-1), nf, th, table)


# speedup vs baseline: 67.6381x; 1.4994x over previous
import functools

import jax
import jax.numpy as jnp
from jax import lax
from jax.experimental import pallas as pl
from jax.experimental.pallas import tpu as pltpu
from jax.experimental.pallas import tpu_sc as plsc

INPUT_DIM = 512
N_CLASSES = 128
MAX_DEPTH = 12
N_NODES = 2**MAX_DEPTH - 1
N_LEAVES = 2**MAX_DEPTH
BATCH = 16384

NC = 2
NS = 16
L = 16
NW = NC * NS
ROWS_PER_W = BATCH // NW
PASS_ROWS = 64
NPASS = ROWS_PER_W // PASS_ROWS
NGRP = PASS_ROWS // L


def _prep_body(sf_ref, lp_ref, nf_ref, table_ref):
    sf = sf_ref[...]
    nf_ref[...] = jnp.clip(jnp.floor(sf), 0, INPUT_DIM - 1).astype(jnp.int32)
    p = lp_ref[...]
    m = jnp.max(p, axis=1, keepdims=True)
    e = jnp.exp(p - m)
    table_ref[...] = e / jnp.sum(e, axis=1, keepdims=True)


def _prep(sf2d, leaf_probabilities):
    return pl.pallas_call(
        _prep_body,
        out_shape=[
            jax.ShapeDtypeStruct((N_LEAVES // 128, 128), jnp.int32),
            jax.ShapeDtypeStruct((N_LEAVES, N_CLASSES), jnp.float32),
        ],
    )(sf2d, leaf_probabilities)


def _traverse_body(
    x_ref, nf_ref, th_ref, table_ref, out_ref,
    nf_v, th_v, xbufs, gidxs, rowbufs, xsems, rsems,
):
    wid = lax.axis_index("s") * NC + lax.axis_index("c")
    base = wid * ROWS_PER_W

    for b in range(2):
        pltpu.async_copy(
            x_ref.at[pl.ds(base + b * PASS_ROWS, PASS_ROWS), :],
            xbufs[b], xsems[b],
        )
    pltpu.sync_copy(nf_ref, nf_v)
    pltpu.sync_copy(th_ref, th_v)

    lane = lax.iota(jnp.int32, L)
    zero = jnp.zeros((L,), jnp.int32)

    def run_pass(b, p, pp):
        pltpu.make_async_copy(
            x_ref.at[pl.ds(base, PASS_ROWS), :], xbufs[b], xsems[b]
        ).wait()
        lrows = [g * L + lane for g in range(NGRP)]
        idxs = [zero] * NGRP
        for _ in range(MAX_DEPTH):
            for g in range(NGRP):
                feat = plsc.load_gather(nf_v, [idxs[g]])
                thr = plsc.load_gather(th_v, [idxs[g]])
                xv = plsc.load_gather(xbufs[b], [lrows[g], feat])
                dec = jnp.where(xv > thr, 1, 0).astype(jnp.int32)
                idxs[g] = idxs[g] * 2 + 1 + dec
        for g in range(NGRP):
            gidxs[b][pl.ds(g * L, L)] = idxs[g] - N_NODES
        pltpu.async_copy(table_ref.at[gidxs[b]], rowbufs[b], rsems[b])
        @pl.when(pp < NPASS // 2 - 1)
        def _():
            pltpu.async_copy(
                x_ref.at[pl.ds(base + (p + 2) * PASS_ROWS, PASS_ROWS), :],
                xbufs[b], xsems[b],
            )
        @pl.when(p >= 1)
        def _():
            pltpu.make_async_copy(
                table_ref.at[gidxs[1 - b]], rowbufs[1 - b], rsems[1 - b]
            ).wait()
            pltpu.sync_copy(
                rowbufs[1 - b],
                out_ref.at[pl.ds(base + (p - 1) * PASS_ROWS, PASS_ROWS), :],
            )

    def body(pp, carry):
        for b in range(2):
            run_pass(b, pp * 2 + b, pp)
        return carry

    lax.fori_loop(0, NPASS // 2, body, 0)

    pltpu.make_async_copy(
        table_ref.at[gidxs[1]], rowbufs[1], rsems[1]
    ).wait()
    pltpu.sync_copy(
        rowbufs[1],
        out_ref.at[pl.ds(base + (NPASS - 1) * PASS_ROWS, PASS_ROWS), :],
    )


@functools.partial(
    pl.kernel,
    out_type=jax.ShapeDtypeStruct((BATCH, N_CLASSES), jnp.float32),
    mesh=plsc.VectorSubcoreMesh(
        core_axis_name="c", subcore_axis_name="s", num_cores=NC,
        num_subcores=NS,
    ),
    scratch_types=[
        pltpu.VMEM((N_LEAVES,), jnp.int32),
        pltpu.VMEM((N_LEAVES,), jnp.float32),
        pltpu.VMEM((PASS_ROWS, INPUT_DIM), jnp.float32),
        pltpu.VMEM((PASS_ROWS, INPUT_DIM), jnp.float32),
        pltpu.VMEM((PASS_ROWS,), jnp.int32),
        pltpu.VMEM((PASS_ROWS,), jnp.int32),
        pltpu.VMEM((PASS_ROWS, N_CLASSES), jnp.float32),
        pltpu.VMEM((PASS_ROWS, N_CLASSES), jnp.float32),
        pltpu.SemaphoreType.DMA,
        pltpu.SemaphoreType.DMA,
        pltpu.SemaphoreType.DMA,
        pltpu.SemaphoreType.DMA,
    ],
    compiler_params=pltpu.CompilerParams(needs_layout_passes=False),
)
def _traverse(
    x, nf, th, table, out,
    nf_v, th_v, xbuf0, xbuf1, gidx0, gidx1, rowbuf0, rowbuf1,
    xsem0, xsem1, rsem0, rsem1,
):
    _traverse_body(
        x, nf, th, table, out,
        nf_v, th_v, (xbuf0, xbuf1), (gidx0, gidx1), (rowbuf0, rowbuf1),
        (xsem0, xsem1), (rsem0, rsem1),
    )


def kernel(x, split_features, split_thresholds, leaf_probabilities):
    sf2d = jnp.pad(split_features, (0, 1)).reshape(N_LEAVES // 128, 128)
    th = jnp.pad(split_thresholds, (0, 1))
    nf2d, table = _prep(sf2d, leaf_probabilities)
    nf = nf2d.reshape(N_LEAVES)
    return _traverse(x, nf, th, table)


# speedup vs baseline: 72.8096x; 1.0765x over previous
import functools

import jax
import jax.numpy as jnp
from jax import lax
from jax.experimental import pallas as pl
from jax.experimental.pallas import tpu as pltpu
from jax.experimental.pallas import tpu_sc as plsc

INPUT_DIM = 512
N_CLASSES = 128
MAX_DEPTH = 12
N_NODES = 2**MAX_DEPTH - 1
N_LEAVES = 2**MAX_DEPTH
BATCH = 16384

NC = 2
NS = 16
L = 16
NW = NC * NS
ROWS_PER_W = BATCH // NW
PASS_ROWS = 64
NPASS = ROWS_PER_W // PASS_ROWS
NGRP = PASS_ROWS // L


def _prep_body(sf_ref, lp_ref, nf_ref, table_ref):
    sf = sf_ref[...]
    nf_ref[...] = jnp.clip(jnp.floor(sf), 0, INPUT_DIM - 1).astype(jnp.int32)
    p = lp_ref[...]
    m = jnp.max(p, axis=1, keepdims=True)
    e = jnp.exp(p - m)
    table_ref[...] = e / jnp.sum(e, axis=1, keepdims=True)


def _prep(split_features, leaf_probabilities):
    return pl.pallas_call(
        _prep_body,
        out_shape=[
            jax.ShapeDtypeStruct((N_NODES,), jnp.int32),
            jax.ShapeDtypeStruct((N_LEAVES, N_CLASSES), jnp.float32),
        ],
    )(split_features, leaf_probabilities)


def _traverse_body(
    x_ref, nf_ref, th_ref, table_ref, out_ref,
    nf_v, th_v, xbufs, gidxs, rowbufs, xsems, rsems,
):
    wid = lax.axis_index("s") * NC + lax.axis_index("c")
    base = wid * ROWS_PER_W

    for b in range(2):
        pltpu.async_copy(
            x_ref.at[pl.ds(base + b * PASS_ROWS, PASS_ROWS), :],
            xbufs[b], xsems[b],
        )
    pltpu.sync_copy(nf_ref, nf_v)
    pltpu.sync_copy(th_ref, th_v)

    lane = lax.iota(jnp.int32, L)
    zero = jnp.zeros((L,), jnp.int32)

    def run_pass(b, p, pp):
        pltpu.make_async_copy(
            x_ref.at[pl.ds(base, PASS_ROWS), :], xbufs[b], xsems[b]
        ).wait()
        lrows = [g * L + lane for g in range(NGRP)]
        idxs = [zero] * NGRP
        for _ in range(MAX_DEPTH):
            for g in range(NGRP):
                feat = plsc.load_gather(nf_v, [idxs[g]])
                thr = plsc.load_gather(th_v, [idxs[g]])
                xv = plsc.load_gather(xbufs[b], [lrows[g], feat])
                dec = jnp.where(xv > thr, 1, 0).astype(jnp.int32)
                idxs[g] = idxs[g] * 2 + 1 + dec
        for g in range(NGRP):
            gidxs[b][pl.ds(g * L, L)] = idxs[g] - N_NODES
        pltpu.async_copy(table_ref.at[gidxs[b]], rowbufs[b], rsems[b])
        @pl.when(pp < NPASS // 2 - 1)
        def _():
            pltpu.async_copy(
                x_ref.at[pl.ds(base + (p + 2) * PASS_ROWS, PASS_ROWS), :],
                xbufs[b], xsems[b],
            )
        @pl.when(p >= 1)
        def _():
            pltpu.make_async_copy(
                table_ref.at[gidxs[1 - b]], rowbufs[1 - b], rsems[1 - b]
            ).wait()
            pltpu.sync_copy(
                rowbufs[1 - b],
                out_ref.at[pl.ds(base + (p - 1) * PASS_ROWS, PASS_ROWS), :],
            )

    def body(pp, carry):
        for b in range(2):
            run_pass(b, pp * 2 + b, pp)
        return carry

    lax.fori_loop(0, NPASS // 2, body, 0)

    pltpu.make_async_copy(
        table_ref.at[gidxs[1]], rowbufs[1], rsems[1]
    ).wait()
    pltpu.sync_copy(
        rowbufs[1],
        out_ref.at[pl.ds(base + (NPASS - 1) * PASS_ROWS, PASS_ROWS), :],
    )


@functools.partial(
    pl.kernel,
    out_type=jax.ShapeDtypeStruct((BATCH, N_CLASSES), jnp.float32),
    mesh=plsc.VectorSubcoreMesh(
        core_axis_name="c", subcore_axis_name="s", num_cores=NC,
        num_subcores=NS,
    ),
    scratch_types=[
        pltpu.VMEM((N_NODES,), jnp.int32),
        pltpu.VMEM((N_NODES,), jnp.float32),
        pltpu.VMEM((PASS_ROWS, INPUT_DIM), jnp.float32),
        pltpu.VMEM((PASS_ROWS, INPUT_DIM), jnp.float32),
        pltpu.VMEM((PASS_ROWS,), jnp.int32),
        pltpu.VMEM((PASS_ROWS,), jnp.int32),
        pltpu.VMEM((PASS_ROWS, N_CLASSES), jnp.float32),
        pltpu.VMEM((PASS_ROWS, N_CLASSES), jnp.float32),
        pltpu.SemaphoreType.DMA,
        pltpu.SemaphoreType.DMA,
        pltpu.SemaphoreType.DMA,
        pltpu.SemaphoreType.DMA,
    ],
    compiler_params=pltpu.CompilerParams(needs_layout_passes=False),
)
def _traverse(
    x, nf, th, table, out,
    nf_v, th_v, xbuf0, xbuf1, gidx0, gidx1, rowbuf0, rowbuf1,
    xsem0, xsem1, rsem0, rsem1,
):
    _traverse_body(
        x, nf, th, table, out,
        nf_v, th_v, (xbuf0, xbuf1), (gidx0, gidx1), (rowbuf0, rowbuf1),
        (xsem0, xsem1), (rsem0, rsem1),
    )


def kernel(x, split_features, split_thresholds, leaf_probabilities):
    nf, table = _prep(split_features, leaf_probabilities)
    return _traverse(x, nf, split_thresholds, table)
